# Initial kernel scaffold; baseline (speedup 1.0000x reference)
import jax
import jax.numpy as jnp
from jax.experimental import pallas as pl


def kernel(X, L, H, C, W_x_i, b_x_i, W_h_i, b_h_i, W_x_f, b_x_f, W_h_f, b_h_f, W_x_c, b_x_c, W_h_c, b_h_c, W_x_o, b_x_o, W_h_o, b_h_o, w_c_i, w_c_f, w_c_o, b_i, b_f, b_c, b_o):
    raise NotImplementedError("write your pallas kernel here")


# speedup vs baseline: 1.5940x; 1.5940x over previous
import functools

import jax
import jax.numpy as jnp
from jax.experimental import pallas as pl
from jax.experimental.pallas import tpu as pltpu

N = 4096
F2 = 128
G4 = 256

BI = 512
BK = 4096
NK = N // BK

def _dot(a, b):
    return jax.lax.dot_general(a.astype(jnp.bfloat16), b.astype(jnp.bfloat16),
                               (((1,), (0,)), ((), ())),
                               preferred_element_type=jnp.float32)


def _pass1_kernel(l_ref, xh_ref, o_ref):
    k = pl.program_id(1)
    p = _dot(l_ref[...], xh_ref[...])

    @pl.when(k == 0)
    def _():
        o_ref[...] = p

    @pl.when(k > 0)
    def _():
        o_ref[...] += p


def _pass2_kernel(l_ref, t1k_ref, xh_ref, t1i_ref, c_ref, w_ref, bcat_ref,
                  wci_ref, wcf_ref, wco_ref, hn_ref, cn_ref, acc_ref):
    k = pl.program_id(1)
    p = _dot(l_ref[...], t1k_ref[...])

    @pl.when(k == 0)
    def _():
        acc_ref[...] = p

    @pl.when(k > 0)
    def _():
        acc_ref[...] += p

    @pl.when(k == NK - 1)
    def _():
        t0 = xh_ref[...]
        t1 = t1i_ref[...]
        t2 = 2.0 * acc_ref[...] - t0
        w = w_ref[...]
        pre = (_dot(t0, w[0]) + _dot(t1, w[1]) + _dot(t2, w[2])
               + bcat_ref[...])
        cin = c_ref[...]
        gi = jax.nn.sigmoid(pre[:, 0:64] + wci_ref[...] * cin)
        gf = jax.nn.sigmoid(pre[:, 64:128] + wcf_ref[...] * cin)
        gt = jnp.tanh(pre[:, 128:192])
        cn = gf * cin + gi * gt
        go = jax.nn.sigmoid(pre[:, 192:256] + wco_ref[...] * cn)
        hn_ref[...] = go * jnp.tanh(cn)
        cn_ref[...] = cn


@functools.partial(jax.jit, static_argnums=())
def _run(XH, L, C, W, bcat, wci, wcf, wco):
    t1 = pl.pallas_call(
        _pass1_kernel,
        grid=(N // BI, NK),
        in_specs=[
            pl.BlockSpec((BI, BK), lambda i, k: (i, k)),
            pl.BlockSpec((BK, F2), lambda i, k: (k, 0)),
        ],
        out_specs=pl.BlockSpec((BI, F2), lambda i, k: (i, 0)),
        out_shape=jax.ShapeDtypeStruct((N, F2), jnp.float32),
        compiler_params=pltpu.CompilerParams(
            dimension_semantics=("parallel", "arbitrary")),
    )(L, XH)

    hn, cn = pl.pallas_call(
        _pass2_kernel,
        grid=(N // BI, NK),
        in_specs=[
            pl.BlockSpec((BI, BK), lambda i, k: (i, k)),
            pl.BlockSpec((BK, F2), lambda i, k: (k, 0)),
            pl.BlockSpec((BI, F2), lambda i, k: (i, 0)),
            pl.BlockSpec((BI, F2), lambda i, k: (i, 0)),
            pl.BlockSpec((BI, 64), lambda i, k: (i, 0)),
            pl.BlockSpec((3, F2, G4), lambda i, k: (0, 0, 0)),
            pl.BlockSpec((1, G4), lambda i, k: (0, 0)),
            pl.BlockSpec((1, 64), lambda i, k: (0, 0)),
            pl.BlockSpec((1, 64), lambda i, k: (0, 0)),
            pl.BlockSpec((1, 64), lambda i, k: (0, 0)),
        ],
        out_specs=[
            pl.BlockSpec((BI, 64), lambda i, k: (i, 0)),
            pl.BlockSpec((BI, 64), lambda i, k: (i, 0)),
        ],
        out_shape=[
            jax.ShapeDtypeStruct((N, 64), jnp.float32),
            jax.ShapeDtypeStruct((N, 64), jnp.float32),
        ],
        scratch_shapes=[pltpu.VMEM((BI, F2), jnp.float32)],
        compiler_params=pltpu.CompilerParams(
            dimension_semantics=("parallel", "arbitrary")),
    )(L, t1, XH, t1, C, W, bcat, wci, wcf, wco)
    return hn, cn


def kernel(X, L, H, C,
           W_x_i, b_x_i, W_h_i, b_h_i,
           W_x_f, b_x_f, W_h_f, b_h_f,
           W_x_c, b_x_c, W_h_c, b_h_c,
           W_x_o, b_x_o, W_h_o, b_h_o,
           w_c_i, w_c_f, w_c_o, b_i, b_f, b_c, b_o):
    XH = jnp.concatenate([X, H], axis=1)
    Wx = jnp.concatenate([W_x_i, W_x_f, W_x_c, W_x_o], axis=2)
    Wh = jnp.concatenate([W_h_i, W_h_f, W_h_c, W_h_o], axis=2)
    W = jnp.concatenate([Wx, Wh], axis=1)
    bcat = jnp.concatenate([
        (b_x_i + b_h_i)[None, :] + b_i,
        (b_x_f + b_h_f)[None, :] + b_f,
        (b_x_c + b_h_c)[None, :] + b_c,
        (b_x_o + b_h_o)[None, :] + b_o,
    ], axis=1)
    return _run(XH, L, C, W, bcat, w_c_i, w_c_f, w_c_o)


# speedup vs baseline: 1.7134x; 1.0749x over previous
import jax
import jax.numpy as jnp
from jax.experimental import pallas as pl
from jax.experimental.pallas import tpu as pltpu

N = 4096
F2 = 128
G4 = 256

BI = 256
NI = N // BI


def _dot(a, b):
    return jax.lax.dot_general(a.astype(jnp.bfloat16), b.astype(jnp.bfloat16),
                               (((1,), (0,)), ((), ())),
                               preferred_element_type=jnp.float32)


def _fused_kernel(l_ref, xh_ref, c_ref, w_ref, bcat_ref,
                  wci_ref, wcf_ref, wco_ref,
                  hn_ref, cn_ref,
                  lbf_ref, t1bf_ref, xhbf_ref):
    p = pl.program_id(0)
    i = pl.program_id(1)
    rows = pl.ds(i * BI, BI)

    @pl.when(p == 0)
    def _():
        @pl.when(i == 0)
        def _():
            xhbf_ref[...] = xh_ref[...].astype(jnp.bfloat16)
        lblk = l_ref[...].astype(jnp.bfloat16)
        lbf_ref[rows, :] = lblk
        t1bf_ref[rows, :] = _dot(lblk, xhbf_ref[...]).astype(jnp.bfloat16)

    @pl.when(p == 1)
    def _():
        lt1 = _dot(lbf_ref[rows, :], t1bf_ref[...])
        t0 = xh_ref[rows, :]
        t2 = 2.0 * lt1 - t0
        w = w_ref[...]
        pre = (_dot(xhbf_ref[rows, :], w[0]) + _dot(t1bf_ref[rows, :], w[1])
               + _dot(t2, w[2]) + bcat_ref[...])
        cin = c_ref[rows, :]
        gi = jax.nn.sigmoid(pre[:, 0:64] + wci_ref[...] * cin)
        gf = jax.nn.sigmoid(pre[:, 64:128] + wcf_ref[...] * cin)
        gt = jnp.tanh(pre[:, 128:192])
        cn = gf * cin + gi * gt
        go = jax.nn.sigmoid(pre[:, 192:256] + wco_ref[...] * cn)
        hn_ref[...] = go * jnp.tanh(cn)
        cn_ref[...] = cn


@jax.jit
def _run(XH, L, C, W, bcat, wci, wcf, wco):
    hn, cn = pl.pallas_call(
        _fused_kernel,
        grid=(2, NI),
        in_specs=[
            pl.BlockSpec((BI, N), lambda p, i: (i + p * (NI - 1 - i), 0)),
            pl.BlockSpec((N, F2), lambda p, i: (0, 0)),
            pl.BlockSpec((N, 64), lambda p, i: (0, 0)),
            pl.BlockSpec((3, F2, G4), lambda p, i: (0, 0, 0)),
            pl.BlockSpec((1, G4), lambda p, i: (0, 0)),
            pl.BlockSpec((1, 64), lambda p, i: (0, 0)),
            pl.BlockSpec((1, 64), lambda p, i: (0, 0)),
            pl.BlockSpec((1, 64), lambda p, i: (0, 0)),
        ],
        out_specs=[
            pl.BlockSpec((BI, 64), lambda p, i: (i * p, 0)),
            pl.BlockSpec((BI, 64), lambda p, i: (i * p, 0)),
        ],
        out_shape=[
            jax.ShapeDtypeStruct((N, 64), jnp.float32),
            jax.ShapeDtypeStruct((N, 64), jnp.float32),
        ],
        scratch_shapes=[
            pltpu.VMEM((N, N), jnp.bfloat16),
            pltpu.VMEM((N, F2), jnp.bfloat16),
            pltpu.VMEM((N, F2), jnp.bfloat16),
        ],
        compiler_params=pltpu.CompilerParams(
            dimension_semantics=("arbitrary", "arbitrary")),
    )(L, XH, C, W, bcat, wci, wcf, wco)
    return hn, cn


def kernel(X, L, H, C,
           W_x_i, b_x_i, W_h_i, b_h_i,
           W_x_f, b_x_f, W_h_f, b_h_f,
           W_x_c, b_x_c, W_h_c, b_h_c,
           W_x_o, b_x_o, W_h_o, b_h_o,
           w_c_i, w_c_f, w_c_o, b_i, b_f, b_c, b_o):
    XH = jnp.concatenate([X, H], axis=1)
    Wx = jnp.concatenate([W_x_i, W_x_f, W_x_c, W_x_o], axis=2)
    Wh = jnp.concatenate([W_h_i, W_h_f, W_h_c, W_h_o], axis=2)
    W = jnp.concatenate([Wx, Wh], axis=1)
    bcat = jnp.concatenate([
        (b_x_i + b_h_i)[None, :] + b_i,
        (b_x_f + b_h_f)[None, :] + b_f,
        (b_x_c + b_h_c)[None, :] + b_c,
        (b_x_o + b_h_o)[None, :] + b_o,
    ], axis=1)
    return _run(XH, L, C, W, bcat, w_c_i, w_c_f, w_c_o)


# speedup vs baseline: 1.9756x; 1.1530x over previous
import jax
import jax.numpy as jnp
from jax.experimental import pallas as pl
from jax.experimental.pallas import tpu as pltpu

N = 4096
F2 = 128
G4 = 256

BI = 512
NI = N // BI


def _dot(a, b):
    return jax.lax.dot_general(a.astype(jnp.bfloat16), b.astype(jnp.bfloat16),
                               (((1,), (0,)), ((), ())),
                               preferred_element_type=jnp.float32)


def _fused_kernel(l_ref, xh_ref, c_ref, w_ref, bcat_ref,
                  wci_ref, wcf_ref, wco_ref,
                  hn_ref, cn_ref,
                  lbf_ref, t1bf_ref, xhbf_ref):
    p = pl.program_id(0)
    i = pl.program_id(1)
    rows = pl.ds(i * BI, BI)

    @pl.when(p == 0)
    def _():
        @pl.when(i == 0)
        def _():
            xhbf_ref[...] = xh_ref[...].astype(jnp.bfloat16)
        lblk = l_ref[...].astype(jnp.bfloat16)
        lbf_ref[rows, :] = lblk
        t1bf_ref[rows, :] = _dot(lblk, xhbf_ref[...]).astype(jnp.bfloat16)

    @pl.when(p == 1)
    def _():
        lt1 = _dot(lbf_ref[rows, :], t1bf_ref[...])
        t0 = xh_ref[rows, :]
        t2 = 2.0 * lt1 - t0
        w = w_ref[...]
        pre = (_dot(xhbf_ref[rows, :], w[0]) + _dot(t1bf_ref[rows, :], w[1])
               + _dot(t2, w[2]) + bcat_ref[...])
        cin = c_ref[rows, :]
        gi = jax.nn.sigmoid(pre[:, 0:64] + wci_ref[...] * cin)
        gf = jax.nn.sigmoid(pre[:, 64:128] + wcf_ref[...] * cin)
        gt = jnp.tanh(pre[:, 128:192])
        cn = gf * cin + gi * gt
        go = jax.nn.sigmoid(pre[:, 192:256] + wco_ref[...] * cn)
        hn_ref[...] = go * jnp.tanh(cn)
        cn_ref[...] = cn


@jax.jit
def _run(XH, L, C, W, bcat, wci, wcf, wco):
    hn, cn = pl.pallas_call(
        _fused_kernel,
        grid=(2, NI),
        in_specs=[
            pl.BlockSpec((BI, N), lambda p, i: (i + p * (NI - 1 - i), 0)),
            pl.BlockSpec((N, F2), lambda p, i: (0, 0)),
            pl.BlockSpec((N, 64), lambda p, i: (0, 0)),
            pl.BlockSpec((3, F2, G4), lambda p, i: (0, 0, 0)),
            pl.BlockSpec((1, G4), lambda p, i: (0, 0)),
            pl.BlockSpec((1, 64), lambda p, i: (0, 0)),
            pl.BlockSpec((1, 64), lambda p, i: (0, 0)),
            pl.BlockSpec((1, 64), lambda p, i: (0, 0)),
        ],
        out_specs=[
            pl.BlockSpec((BI, 64), lambda p, i: (i * p, 0)),
            pl.BlockSpec((BI, 64), lambda p, i: (i * p, 0)),
        ],
        out_shape=[
            jax.ShapeDtypeStruct((N, 64), jnp.float32),
            jax.ShapeDtypeStruct((N, 64), jnp.float32),
        ],
        scratch_shapes=[
            pltpu.VMEM((N, N), jnp.bfloat16),
            pltpu.VMEM((N, F2), jnp.bfloat16),
            pltpu.VMEM((N, F2), jnp.bfloat16),
        ],
        compiler_params=pltpu.CompilerParams(
            dimension_semantics=("arbitrary", "arbitrary")),
    )(L, XH, C, W, bcat, wci, wcf, wco)
    return hn, cn


def kernel(X, L, H, C,
           W_x_i, b_x_i, W_h_i, b_h_i,
           W_x_f, b_x_f, W_h_f, b_h_f,
           W_x_c, b_x_c, W_h_c, b_h_c,
           W_x_o, b_x_o, W_h_o, b_h_o,
           w_c_i, w_c_f, w_c_o, b_i, b_f, b_c, b_o):
    XH = jnp.concatenate([X, H], axis=1)
    Wx = jnp.concatenate([W_x_i, W_x_f, W_x_c, W_x_o], axis=2)
    Wh = jnp.concatenate([W_h_i, W_h_f, W_h_c, W_h_o], axis=2)
    W = jnp.concatenate([Wx, Wh], axis=1)
    bcat = jnp.concatenate([
        (b_x_i + b_h_i)[None, :] + b_i,
        (b_x_f + b_h_f)[None, :] + b_f,
        (b_x_c + b_h_c)[None, :] + b_c,
        (b_x_o + b_h_o)[None, :] + b_o,
    ], axis=1)
    return _run(XH, L, C, W, bcat, w_c_i, w_c_f, w_c_o)
